# baseline (speedup 1.0000x reference)
import functools

import jax
import jax.numpy as jnp
from jax import lax
from jax.experimental import pallas as pl
from jax.experimental.pallas import tpu as pltpu
from jax.experimental.pallas import tpu_sc as plsc

B = 4
S = 4096
PE_DIM = 2048
NC = 2
NS = 16
NW = NC * NS
TILES_PER_BATCH = NW // B
ROWS_PER_W = S // TILES_PER_BATCH
CHUNK = 16
NCHUNKS = ROWS_PER_W // CHUNK
LANES = 16
COLS = PE_DIM // LANES


def _sc_body(x_hbm, pe_hbm, out_hbm, xbuf, pebuf):
    wid = lax.axis_index("s") * NC + lax.axis_index("c")
    b = wid // TILES_PER_BATCH
    s_base = (wid % TILES_PER_BATCH) * ROWS_PER_W

    def chunk_body(i, _):
        s0 = s_base + i * CHUNK
        pltpu.sync_copy(x_hbm.at[b, pl.ds(s0, CHUNK), :], xbuf)
        pltpu.sync_copy(pe_hbm.at[pl.ds(s0, CHUNK), :], pebuf)

        def row_body(r, _):
            for c in range(COLS):
                v = pebuf[r, pl.ds(c * LANES, LANES)]
                plsc.addupdate(xbuf.at[r, pl.ds(c * LANES, LANES)], v)
            return 0

        lax.fori_loop(0, CHUNK, row_body, 0)
        pltpu.sync_copy(xbuf, out_hbm.at[b, pl.ds(s0, CHUNK), :])
        return 0

    lax.fori_loop(0, NCHUNKS, chunk_body, 0)


_mesh = plsc.VectorSubcoreMesh(
    core_axis_name="c", subcore_axis_name="s", num_cores=NC, num_subcores=NS
)

_sc_call = functools.partial(
    pl.kernel,
    out_type=jax.ShapeDtypeStruct((B, S, PE_DIM), jnp.float32),
    mesh=_mesh,
    scratch_types=[
        pltpu.VMEM((CHUNK, PE_DIM), jnp.float32),
        pltpu.VMEM((CHUNK, PE_DIM), jnp.float32),
    ],
)(_sc_body)


def kernel(x, emb_weight):
    return _sc_call(x, emb_weight)


# speedup vs baseline: 1.7361x; 1.7361x over previous
import functools

import jax
import jax.numpy as jnp
from jax import lax
from jax.experimental import pallas as pl
from jax.experimental.pallas import tpu as pltpu
from jax.experimental.pallas import tpu_sc as plsc

B = 4
S = 4096
PE_DIM = 2048
NC = 2
NS = 16
NW = NC * NS
TILES_PER_BATCH = NW // B
ROWS_PER_W = S // TILES_PER_BATCH
CHUNK = 8
NCHUNKS = ROWS_PER_W // CHUNK
NPAIRS = NCHUNKS // 2
LANES = 16
COLS = PE_DIM // LANES


def _add_chunk(xbuf, pebuf):
    def row_body(r, _):
        for c in range(COLS):
            v = pebuf[r, pl.ds(c * LANES, LANES)]
            plsc.addupdate(xbuf.at[r, pl.ds(c * LANES, LANES)], v)
        return 0

    lax.fori_loop(0, CHUNK, row_body, 0)


def _sc_body(x_hbm, pe_hbm, out_hbm, xb0, pb0, xb1, pb1,
             six0, sip0, six1, sip1, so0, so1):
    wid = lax.axis_index("s") * NC + lax.axis_index("c")
    b = wid // TILES_PER_BATCH
    s_base = (wid % TILES_PER_BATCH) * ROWS_PER_W

    xbufs = (xb0, xb1)
    pebufs = (pb0, pb1)
    sin_x = (six0, six1)
    sin_p = (sip0, sip1)
    souts = (so0, so1)

    def start_in(g, k):
        s0 = s_base + g * CHUNK
        pltpu.async_copy(x_hbm.at[b, pl.ds(s0, CHUNK), :], xbufs[k], sin_x[k])
        pltpu.async_copy(pe_hbm.at[pl.ds(s0, CHUNK), :], pebufs[k], sin_p[k])

    def wait_in(k):
        pltpu.make_async_copy(x_hbm.at[b, pl.ds(s_base, CHUNK), :], xbufs[k], sin_x[k]).wait()
        pltpu.make_async_copy(pe_hbm.at[pl.ds(s_base, CHUNK), :], pebufs[k], sin_p[k]).wait()

    def start_out(g, k):
        s0 = s_base + g * CHUNK
        pltpu.async_copy(xbufs[k], out_hbm.at[b, pl.ds(s0, CHUNK), :], souts[k])

    def wait_out(k):
        pltpu.make_async_copy(xbufs[k], out_hbm.at[b, pl.ds(s_base, CHUNK), :], souts[k]).wait()

    start_in(0, 0)
    start_in(1, 1)

    def pair_body(p, _):
        g0 = 2 * p
        wait_in(0)
        _add_chunk(xb0, pb0)
        start_out(g0, 0)
        @pl.when(p + 1 < NPAIRS)
        def _():
            wait_out(0)
            start_in(g0 + 2, 0)
        wait_in(1)
        _add_chunk(xb1, pb1)
        start_out(g0 + 1, 1)
        @pl.when(p + 1 < NPAIRS)
        def _():
            wait_out(1)
            start_in(g0 + 3, 1)
        return 0

    lax.fori_loop(0, NPAIRS, pair_body, 0)
    wait_out(0)
    wait_out(1)


_mesh = plsc.VectorSubcoreMesh(
    core_axis_name="c", subcore_axis_name="s", num_cores=NC, num_subcores=NS
)

_sc_call = functools.partial(
    pl.kernel,
    out_type=jax.ShapeDtypeStruct((B, S, PE_DIM), jnp.float32),
    mesh=_mesh,
    scratch_types=[
        pltpu.VMEM((CHUNK, PE_DIM), jnp.float32),
        pltpu.VMEM((CHUNK, PE_DIM), jnp.float32),
        pltpu.VMEM((CHUNK, PE_DIM), jnp.float32),
        pltpu.VMEM((CHUNK, PE_DIM), jnp.float32),
        pltpu.SemaphoreType.DMA,
        pltpu.SemaphoreType.DMA,
        pltpu.SemaphoreType.DMA,
        pltpu.SemaphoreType.DMA,
        pltpu.SemaphoreType.DMA,
        pltpu.SemaphoreType.DMA,
    ],
)(_sc_body)


def kernel(x, emb_weight):
    return _sc_call(x, emb_weight)


# speedup vs baseline: 1.8319x; 1.0552x over previous
import functools

import jax
import jax.numpy as jnp
from jax import lax
from jax.experimental import pallas as pl
from jax.experimental.pallas import tpu as pltpu
from jax.experimental.pallas import tpu_sc as plsc

B = 4
S = 4096
PE_DIM = 2048
NC = 2
NS = 16
NW = NC * NS
TILES_PER_BATCH = NW // B
ROWS_PER_W = S // TILES_PER_BATCH
CHUNK = 8
NCHUNKS = ROWS_PER_W // CHUNK
NPAIRS = NCHUNKS // 2
LANES = 16
COLS = PE_DIM // LANES


def _add_chunk(xbuf, pebuf):
    @plsc.parallel_loop(0, CHUNK, unroll=2)
    def _(r):
        for c in range(COLS):
            v = pebuf[r, pl.ds(c * LANES, LANES)]
            plsc.addupdate(xbuf.at[r, pl.ds(c * LANES, LANES)], v)


def _sc_body(x_hbm, pe_hbm, out_hbm, xb0, pb0, xb1, pb1,
             six0, sip0, six1, sip1, so0, so1):
    wid = lax.axis_index("s") * NC + lax.axis_index("c")
    b = wid // TILES_PER_BATCH
    s_base = (wid % TILES_PER_BATCH) * ROWS_PER_W

    xbufs = (xb0, xb1)
    pebufs = (pb0, pb1)
    sin_x = (six0, six1)
    sin_p = (sip0, sip1)
    souts = (so0, so1)

    def start_in(g, k):
        s0 = s_base + g * CHUNK
        pltpu.async_copy(x_hbm.at[b, pl.ds(s0, CHUNK), :], xbufs[k], sin_x[k])
        pltpu.async_copy(pe_hbm.at[pl.ds(s0, CHUNK), :], pebufs[k], sin_p[k])

    def wait_in(k):
        pltpu.make_async_copy(x_hbm.at[b, pl.ds(s_base, CHUNK), :], xbufs[k], sin_x[k]).wait()
        pltpu.make_async_copy(pe_hbm.at[pl.ds(s_base, CHUNK), :], pebufs[k], sin_p[k]).wait()

    def start_out(g, k):
        s0 = s_base + g * CHUNK
        pltpu.async_copy(xbufs[k], out_hbm.at[b, pl.ds(s0, CHUNK), :], souts[k])

    def wait_out(k):
        pltpu.make_async_copy(xbufs[k], out_hbm.at[b, pl.ds(s_base, CHUNK), :], souts[k]).wait()

    start_in(0, 0)
    start_in(1, 1)

    def pair_body(p, _):
        g0 = 2 * p
        wait_in(0)
        _add_chunk(xb0, pb0)
        start_out(g0, 0)
        @pl.when(p + 1 < NPAIRS)
        def _():
            wait_out(0)
            start_in(g0 + 2, 0)
        wait_in(1)
        _add_chunk(xb1, pb1)
        start_out(g0 + 1, 1)
        @pl.when(p + 1 < NPAIRS)
        def _():
            wait_out(1)
            start_in(g0 + 3, 1)
        return 0

    lax.fori_loop(0, NPAIRS, pair_body, 0)
    wait_out(0)
    wait_out(1)


_mesh = plsc.VectorSubcoreMesh(
    core_axis_name="c", subcore_axis_name="s", num_cores=NC, num_subcores=NS
)

_sc_call = functools.partial(
    pl.kernel,
    out_type=jax.ShapeDtypeStruct((B, S, PE_DIM), jnp.float32),
    mesh=_mesh,
    scratch_types=[
        pltpu.VMEM((CHUNK, PE_DIM), jnp.float32),
        pltpu.VMEM((CHUNK, PE_DIM), jnp.float32),
        pltpu.VMEM((CHUNK, PE_DIM), jnp.float32),
        pltpu.VMEM((CHUNK, PE_DIM), jnp.float32),
        pltpu.SemaphoreType.DMA,
        pltpu.SemaphoreType.DMA,
        pltpu.SemaphoreType.DMA,
        pltpu.SemaphoreType.DMA,
        pltpu.SemaphoreType.DMA,
        pltpu.SemaphoreType.DMA,
    ],
)(_sc_body)


def kernel(x, emb_weight):
    return _sc_call(x, emb_weight)


# speedup vs baseline: 4.5154x; 2.4649x over previous
import jax
import jax.numpy as jnp
from jax.experimental import pallas as pl
from jax.experimental.pallas import tpu as pltpu

BS = 1024


def _add_kernel(x_ref, pe_ref, out_ref):
    out_ref[0, :, :] = x_ref[0, :, :] + pe_ref[:, :]


def kernel(x, emb_weight):
    b, s, d = x.shape
    grid = (s // BS, b)
    return pl.pallas_call(
        _add_kernel,
        grid=grid,
        in_specs=[
            pl.BlockSpec((1, BS, d), lambda i, j: (j, i, 0)),
            pl.BlockSpec((BS, d), lambda i, j: (i, 0)),
        ],
        out_specs=pl.BlockSpec((1, BS, d), lambda i, j: (j, i, 0)),
        out_shape=jax.ShapeDtypeStruct((b, s, d), x.dtype),
        compiler_params=pltpu.CompilerParams(
            dimension_semantics=("parallel", "arbitrary"),
        ),
    )(x, emb_weight)
